# baseline (speedup 1.0000x reference)
import functools

import jax
import jax.numpy as jnp
from jax.experimental import pallas as pl

HIDDEN_DIM = 4096
NUM_EXPERTS = 16
NUM_TOKENS = 16384
THRESHOLD = 0.5
BLOCK_M = 512


def _gate_body(x_ref, w_ref, logits_ref, mask_ref):
    logits = jnp.dot(x_ref[...], w_ref[...], preferred_element_type=jnp.float32)
    logits_ref[...] = logits
    gate = jax.nn.sigmoid(logits)
    mask_ref[...] = jnp.where(gate > THRESHOLD, 1, 0).astype(jnp.int32)


@jax.jit
def kernel(cls_hidden_states, W_gate):
    m, k = cls_hidden_states.shape
    n = W_gate.shape[1]
    grid = (m // BLOCK_M,)
    return pl.pallas_call(
        _gate_body,
        grid=grid,
        in_specs=[
            pl.BlockSpec((BLOCK_M, k), lambda i: (i, 0)),
            pl.BlockSpec((k, n), lambda i: (0, 0)),
        ],
        out_specs=[
            pl.BlockSpec((BLOCK_M, n), lambda i: (i, 0)),
            pl.BlockSpec((BLOCK_M, n), lambda i: (i, 0)),
        ],
        out_shape=[
            jax.ShapeDtypeStruct((m, n), jnp.float32),
            jax.ShapeDtypeStruct((m, n), jnp.int32),
        ],
    )(cls_hidden_states, W_gate)


# speedup vs baseline: 1.0371x; 1.0371x over previous
import functools

import jax
import jax.numpy as jnp
from jax.experimental import pallas as pl
from jax.experimental.pallas import tpu as pltpu

HIDDEN_DIM = 4096
NUM_EXPERTS = 16
NUM_TOKENS = 16384
THRESHOLD = 0.5
BLOCK_M = 1024


def _gate_body(x_ref, w_ref, logits_ref, mask_ref):
    logits = jnp.dot(x_ref[...], w_ref[...], preferred_element_type=jnp.float32)
    logits_ref[...] = logits
    gate = jax.nn.sigmoid(logits)
    mask_ref[...] = jnp.where(gate > THRESHOLD, 1, 0).astype(jnp.int32)


@jax.jit
def kernel(cls_hidden_states, W_gate):
    m, k = cls_hidden_states.shape
    n = W_gate.shape[1]
    grid = (m // BLOCK_M,)
    return pl.pallas_call(
        _gate_body,
        grid=grid,
        in_specs=[
            pl.BlockSpec((BLOCK_M, k), lambda i: (i, 0)),
            pl.BlockSpec((k, n), lambda i: (0, 0)),
        ],
        out_specs=[
            pl.BlockSpec((BLOCK_M, n), lambda i: (i, 0)),
            pl.BlockSpec((BLOCK_M, n), lambda i: (i, 0)),
        ],
        out_shape=[
            jax.ShapeDtypeStruct((m, n), jnp.float32),
            jax.ShapeDtypeStruct((m, n), jnp.int32),
        ],
        compiler_params=pltpu.CompilerParams(
            dimension_semantics=("parallel",),
        ),
    )(cls_hidden_states, W_gate)
